# baseline (speedup 1.0000x reference)
import jax
import jax.numpy as jnp
from jax import lax
from jax.experimental import pallas as pl
from jax.experimental.pallas import tpu as pltpu
from jax.experimental.pallas import tpu_sc as plsc

B = 1024
D = 512
N_PREFIX = 2
N_MIDDLE = 2
N_ATTR = 16
N_SUFFIX = 56
T = N_PREFIX + 1 + N_MIDDLE + N_ATTR + N_SUFFIX

OFF_S = N_PREFIX * D
OFF_MID = (N_PREFIX + 1) * D
OFF_ATTR = (N_PREFIX + 1 + N_MIDDLE) * D
OFF_SUF = (N_PREFIX + 1 + N_MIDDLE + N_ATTR) * D
ROW = T * D

_info = plsc.get_sparse_core_info()
_NC = _info.num_cores
_NS = _info.num_subcores
NW = _NC * _NS
RPW = B // NW


def _body(s_ref, attr_ref, pref_ref, mid_ref, suf_ref, out_ref, buf):
    wid = lax.axis_index("s") * _NC + lax.axis_index("c")
    base = wid * RPW
    pltpu.sync_copy(pref_ref, buf.at[pl.ds(0, N_PREFIX * D)])
    pltpu.sync_copy(mid_ref, buf.at[pl.ds(OFF_MID, N_MIDDLE * D)])
    pltpu.sync_copy(suf_ref, buf.at[pl.ds(OFF_SUF, N_SUFFIX * D)])

    def step(i, carry):
        b = base + i
        pltpu.sync_copy(s_ref.at[b], buf.at[pl.ds(OFF_S, D)])
        pltpu.sync_copy(attr_ref.at[b], buf.at[pl.ds(OFF_ATTR, N_ATTR * D)])
        pltpu.sync_copy(buf, out_ref.at[b])
        return carry

    lax.fori_loop(0, RPW, step, 0)


def kernel(s_star, attr_tokens, token_prefix, token_middle, token_suffix):
    mesh = plsc.VectorSubcoreMesh(core_axis_name="c", subcore_axis_name="s")
    f = pl.kernel(
        _body,
        out_type=jax.ShapeDtypeStruct((B, ROW), jnp.float32),
        mesh=mesh,
        scratch_types=[pltpu.VMEM((ROW,), jnp.float32)],
    )
    flat = f(
        s_star,
        attr_tokens.reshape(B, N_ATTR * D),
        token_prefix.reshape(N_PREFIX * D),
        token_middle.reshape(N_MIDDLE * D),
        token_suffix.reshape(N_SUFFIX * D),
    )
    return flat.reshape(B, T, D)


# speedup vs baseline: 1.0707x; 1.0707x over previous
import jax
import jax.numpy as jnp
from jax import lax
from jax.experimental import pallas as pl
from jax.experimental.pallas import tpu as pltpu
from jax.experimental.pallas import tpu_sc as plsc

B = 1024
D = 512
N_PREFIX = 2
N_MIDDLE = 2
N_ATTR = 16
N_SUFFIX = 56
T = N_PREFIX + 1 + N_MIDDLE + N_ATTR + N_SUFFIX

OFF_S = N_PREFIX * D
OFF_MID = (N_PREFIX + 1) * D
OFF_ATTR = (N_PREFIX + 1 + N_MIDDLE) * D
OFF_SUF = (N_PREFIX + 1 + N_MIDDLE + N_ATTR) * D
ROW = T * D

_info = plsc.get_sparse_core_info()
_NC = _info.num_cores
_NS = _info.num_subcores
NW = _NC * _NS
RPW = B // NW


def _body(s_ref, attr_ref, pref_ref, mid_ref, suf_ref, out_ref,
          bufA, bufB, sA_s, sA_a, sA_o, sB_s, sB_a, sB_o):
    wid = lax.axis_index("s") * _NC + lax.axis_index("c")
    base = wid * RPW

    def stage_consts(buf):
        pltpu.sync_copy(pref_ref, buf.at[pl.ds(0, N_PREFIX * D)])
        pltpu.sync_copy(mid_ref, buf.at[pl.ds(OFF_MID, N_MIDDLE * D)])
        pltpu.sync_copy(suf_ref, buf.at[pl.ds(OFF_SUF, N_SUFFIX * D)])

    def start_in(b, buf, ssem, asem):
        pltpu.make_async_copy(s_ref.at[b], buf.at[pl.ds(OFF_S, D)], ssem).start()
        pltpu.make_async_copy(
            attr_ref.at[b], buf.at[pl.ds(OFF_ATTR, N_ATTR * D)], asem).start()

    def wait_in(buf, ssem, asem):
        pltpu.make_async_copy(s_ref.at[0], buf.at[pl.ds(OFF_S, D)], ssem).wait()
        pltpu.make_async_copy(
            attr_ref.at[0], buf.at[pl.ds(OFF_ATTR, N_ATTR * D)], asem).wait()

    def start_out(b, buf, osem):
        pltpu.make_async_copy(buf, out_ref.at[b], osem).start()

    def wait_out(buf, osem):
        pltpu.make_async_copy(buf, out_ref.at[0], osem).wait()

    stage_consts(bufA)
    stage_consts(bufB)
    start_in(base, bufA, sA_s, sA_a)
    start_in(base + 1, bufB, sB_s, sB_a)
    wait_in(bufA, sA_s, sA_a)
    start_out(base, bufA, sA_o)
    wait_in(bufB, sB_s, sB_a)
    start_out(base + 1, bufB, sB_o)

    def pair(j, carry):
        r0 = base + 2 * j
        wait_out(bufA, sA_o)
        start_in(r0, bufA, sA_s, sA_a)
        wait_out(bufB, sB_o)
        start_in(r0 + 1, bufB, sB_s, sB_a)
        wait_in(bufA, sA_s, sA_a)
        start_out(r0, bufA, sA_o)
        wait_in(bufB, sB_s, sB_a)
        start_out(r0 + 1, bufB, sB_o)
        return carry

    lax.fori_loop(1, RPW // 2, pair, 0)
    wait_out(bufA, sA_o)
    wait_out(bufB, sB_o)


def kernel(s_star, attr_tokens, token_prefix, token_middle, token_suffix):
    mesh = plsc.VectorSubcoreMesh(core_axis_name="c", subcore_axis_name="s")
    f = pl.kernel(
        _body,
        out_type=jax.ShapeDtypeStruct((B, ROW), jnp.float32),
        mesh=mesh,
        scratch_types=[
            pltpu.VMEM((ROW,), jnp.float32),
            pltpu.VMEM((ROW,), jnp.float32),
            pltpu.SemaphoreType.DMA,
            pltpu.SemaphoreType.DMA,
            pltpu.SemaphoreType.DMA,
            pltpu.SemaphoreType.DMA,
            pltpu.SemaphoreType.DMA,
            pltpu.SemaphoreType.DMA,
        ],
    )
    flat = f(
        s_star,
        attr_tokens.reshape(B, N_ATTR * D),
        token_prefix.reshape(N_PREFIX * D),
        token_middle.reshape(N_MIDDLE * D),
        token_suffix.reshape(N_SUFFIX * D),
    )
    return flat.reshape(B, T, D)


# speedup vs baseline: 1.5081x; 1.4085x over previous
import jax
import jax.numpy as jnp
from jax import lax
from jax.experimental import pallas as pl
from jax.experimental.pallas import tpu as pltpu
from jax.experimental.pallas import tpu_sc as plsc

B = 1024
D = 512
N_PREFIX = 2
N_MIDDLE = 2
N_ATTR = 16
N_SUFFIX = 56
T = N_PREFIX + 1 + N_MIDDLE + N_ATTR + N_SUFFIX

OFF_S = N_PREFIX
OFF_MID = N_PREFIX + 1
OFF_ATTR = OFF_MID + N_MIDDLE
OFF_SUF = OFF_ATTR + N_ATTR

_info = plsc.get_sparse_core_info()
_NC = _info.num_cores
_NS = _info.num_subcores
NW = _NC * _NS
RPW = B // NW


def _body(s_ref, attr_ref, pref_ref, mid_ref, suf_ref, out_ref,
          bufA, bufB, sA_s, sA_a, sA_o, sB_s, sB_a, sB_o):
    cid = lax.axis_index("c")
    sid = lax.axis_index("s")
    base = (sid * _NC + cid) * RPW
    z = lax.div(sid, _NS)

    def stage_consts(buf):
        for t in range(N_PREFIX):
            pltpu.sync_copy(pref_ref.at[t], buf.at[z + t])
        for t in range(N_MIDDLE):
            pltpu.sync_copy(mid_ref.at[t], buf.at[z + OFF_MID + t])
        for t in range(N_SUFFIX):
            pltpu.sync_copy(suf_ref.at[t], buf.at[z + OFF_SUF + t])

    def start_in(b, buf, ssem, asem):
        pltpu.make_async_copy(s_ref.at[b], buf.at[z + OFF_S], ssem).start()
        for t in range(N_ATTR):
            pltpu.make_async_copy(
                attr_ref.at[b * N_ATTR + t], buf.at[z + OFF_ATTR + t], asem).start()

    def wait_in(buf, ssem, asem):
        pltpu.make_async_copy(s_ref.at[0], buf.at[z + OFF_S], ssem).wait()
        for t in range(N_ATTR):
            pltpu.make_async_copy(
                attr_ref.at[t], buf.at[z + OFF_ATTR + t], asem).wait()

    def start_out(b, buf, osem):
        pltpu.make_async_copy(buf, out_ref.at[b], osem).start()

    def wait_out(buf, osem):
        pltpu.make_async_copy(buf, out_ref.at[0], osem).wait()

    stage_consts(bufA)
    stage_consts(bufB)
    start_in(base, bufA, sA_s, sA_a)
    start_in(base + 1, bufB, sB_s, sB_a)
    wait_in(bufA, sA_s, sA_a)
    start_out(base, bufA, sA_o)
    wait_in(bufB, sB_s, sB_a)
    start_out(base + 1, bufB, sB_o)

    def pair(j, carry):
        r0 = base + 2 * j
        wait_out(bufA, sA_o)
        start_in(r0, bufA, sA_s, sA_a)
        wait_out(bufB, sB_o)
        start_in(r0 + 1, bufB, sB_s, sB_a)
        wait_in(bufA, sA_s, sA_a)
        start_out(r0, bufA, sA_o)
        wait_in(bufB, sB_s, sB_a)
        start_out(r0 + 1, bufB, sB_o)
        return carry

    lax.fori_loop(1, RPW // 2, pair, 0)
    wait_out(bufA, sA_o)
    wait_out(bufB, sB_o)


def kernel(s_star, attr_tokens, token_prefix, token_middle, token_suffix):
    mesh = plsc.VectorSubcoreMesh(core_axis_name="c", subcore_axis_name="s")
    f = pl.kernel(
        _body,
        out_type=jax.ShapeDtypeStruct((B, T, D), jnp.float32),
        mesh=mesh,
        scratch_types=[
            pltpu.VMEM((T, D), jnp.float32),
            pltpu.VMEM((T, D), jnp.float32),
            pltpu.SemaphoreType.DMA,
            pltpu.SemaphoreType.DMA,
            pltpu.SemaphoreType.DMA,
            pltpu.SemaphoreType.DMA,
            pltpu.SemaphoreType.DMA,
            pltpu.SemaphoreType.DMA,
        ],
    )
    return f(
        s_star,
        attr_tokens.reshape(B * N_ATTR, D),
        token_prefix.reshape(N_PREFIX, D),
        token_middle.reshape(N_MIDDLE, D),
        token_suffix.reshape(N_SUFFIX, D),
    )


# speedup vs baseline: 2.6157x; 1.7345x over previous
import jax
import jax.numpy as jnp
from jax import lax
from jax.experimental import pallas as pl
from jax.experimental.pallas import tpu as pltpu
from jax.experimental.pallas import tpu_sc as plsc

B = 1024
D = 512
N_PREFIX = 2
N_MIDDLE = 2
N_ATTR = 16
N_SUFFIX = 56
T = N_PREFIX + 1 + N_MIDDLE + N_ATTR + N_SUFFIX
N_CONST = N_PREFIX + N_MIDDLE + N_SUFFIX

OFF_S = 2
OFF_ATTR = 5
OFF_SUF = 21

_info = plsc.get_sparse_core_info()
_NC = _info.num_cores
_NS = _info.num_subcores
NW = _NC * _NS

QB = 256
NQ = B // QB
M_TOTAL = T * NQ
GR = 64


def _body(s_ref, attr_ref, const_ref, out_ref,
          rep_v, g0, g1, idx_v, osem, gsem):
    cid = lax.axis_index("c")
    sid = lax.axis_index("s")
    wid = sid * _NC + cid
    m0 = wid * M_TOTAL // NW
    m1 = (wid + 1) * M_TOTAL // NW
    gbufs = (g0, g1)

    def wait_outs(n, rows):
        for _ in range(n):
            pltpu.make_async_copy(
                out_ref.at[pl.ds(0, rows)], out_ref.at[pl.ds(0, rows)],
                osem).wait()

    def do_const(t, b0, last_r):
        r = jnp.where(t < OFF_S, t,
                      jnp.where(t < OFF_ATTR, t - 1, t - OFF_ATTR))

        def refill():
            for i in range(32):
                pltpu.make_async_copy(const_ref.at[r], rep_v.at[i], gsem).start()
            for i in range(32):
                pltpu.make_async_copy(const_ref.at[0], rep_v.at[0], gsem).wait()
            return r

        lax.cond(r != last_r, refill, lambda: r)
        dst0 = t * B + b0
        for k in range(QB // 32):
            pltpu.make_async_copy(
                rep_v, out_ref.at[pl.ds(dst0 + 32 * k, 32)], osem).start()
        wait_outs(QB // 32, 32)
        return r

    def do_s(b0):
        pltpu.sync_copy(
            s_ref.at[pl.ds(b0, QB)], out_ref.at[pl.ds(OFF_S * B + b0, QB)])

    def do_attr(t, b0):
        j = t - OFF_ATTR
        it = lax.iota(jnp.int32, 16)
        for p in range(QB // GR):
            g = gbufs[p % 2]
            if p >= 2:
                wait_outs(1, GR)
            for k in range(GR // 16):
                idx_v[pl.ds(16 * k, 16)] = (
                    it + (b0 + GR * p + 16 * k)) * N_ATTR + j
            pltpu.async_copy(attr_ref.at[idx_v], g, gsem).wait()
            pltpu.make_async_copy(
                g, out_ref.at[pl.ds(t * B + b0 + GR * p, GR)], osem).start()
        wait_outs(2, GR)

    def step(m, last_r):
        t = m // NQ
        b0 = (m % NQ) * QB

        def s_br():
            do_s(b0)
            return last_r

        def attr_br():
            do_attr(t, b0)
            return last_r

        def const_br():
            return do_const(t, b0, last_r)

        return lax.cond(
            t == OFF_S,
            s_br,
            lambda: lax.cond(
                jnp.logical_and(t >= OFF_ATTR, t < OFF_SUF),
                attr_br, const_br),
        )

    lax.fori_loop(m0, m1, step, jnp.int32(-1))


def kernel(s_star, attr_tokens, token_prefix, token_middle, token_suffix):
    consts = jnp.concatenate(
        [token_prefix.reshape(N_PREFIX, D),
         token_middle.reshape(N_MIDDLE, D),
         token_suffix.reshape(N_SUFFIX, D)], axis=0)
    mesh = plsc.VectorSubcoreMesh(core_axis_name="c", subcore_axis_name="s")
    f = pl.kernel(
        _body,
        out_type=jax.ShapeDtypeStruct((T * B, D), jnp.float32),
        mesh=mesh,
        scratch_types=[
            pltpu.VMEM((32, D), jnp.float32),
            pltpu.VMEM((GR, D), jnp.float32),
            pltpu.VMEM((GR, D), jnp.float32),
            pltpu.VMEM((GR,), jnp.int32),
            pltpu.SemaphoreType.DMA,
            pltpu.SemaphoreType.DMA,
        ],
    )
    flat = f(s_star, attr_tokens.reshape(B * N_ATTR, D), consts)
    return flat.reshape(T, B, D).transpose(1, 0, 2)
